# baseline (speedup 1.0000x reference)
import functools
import numpy as np
import jax
import jax.numpy as jnp
from jax import lax
from jax.experimental import pallas as pl
from jax.experimental.pallas import tpu as pltpu
from jax.experimental.pallas import tpu_sc as plsc

L = 16
T = 2 ** 19
N_RAYS = 4096
N = 262144
P1 = np.int32(np.uint32(2654435761).view(np.int32))
P2 = np.int32(805459861)
IMASK = np.int32(T - 1)
RES = [float(np.floor(16.0 * 1.3819 ** l)) for l in range(L)]

NW = 32
NT = N // NW
C = 256
NCH = NT // C
NG = C // 16
NEG = -3.4e38


def _hash_encode_sc(xs, ys, zs, tab1, resf):
    mesh = plsc.VectorSubcoreMesh(core_axis_name="c", subcore_axis_name="s")

    @functools.partial(
        pl.kernel, mesh=mesh,
        out_type=jax.ShapeDtypeStruct((2 * L * N,), jnp.float32),
        scratch_types=[
            pltpu.VMEM((C,), jnp.float32),
            pltpu.VMEM((C,), jnp.float32),
            pltpu.VMEM((C,), jnp.float32),
            pltpu.VMEM((C,), jnp.float32),
            pltpu.VMEM((C,), jnp.float32),
            pltpu.VMEM((C,), jnp.float32),
            pltpu.VMEM((16 * C,), jnp.int32),
            pltpu.VMEM((16 * C,), jnp.float32),
            pltpu.VMEM((2 * L, C), jnp.float32),
            pltpu.VMEM((L,), jnp.float32),
            pltpu.SemaphoreType.DMA,
            pltpu.SemaphoreType.DMA,
        ],
        compiler_params=pltpu.CompilerParams(needs_layout_passes=False),
    )
    def k(xs_h, ys_h, zs_h, tab_h, res_h, out_h,
          xv, yv, zv, fx, fy, fz, idxb, rows, hbuf, resv, sem, osem):
        wid = lax.axis_index("s") * 2 + lax.axis_index("c")
        tbase = wid * NT
        pltpu.sync_copy(res_h, resv)

        def chunk_body(ci, _):
            base = tbase + ci * C
            pltpu.sync_copy(xs_h.at[pl.ds(base, C)], xv)
            pltpu.sync_copy(ys_h.at[pl.ds(base, C)], yv)
            pltpu.sync_copy(zs_h.at[pl.ds(base, C)], zv)

            def level_body(l, _):
                res = plsc.load_gather(resv, [jnp.full((16,), l, jnp.int32)])
                lbase2 = l * (2 * T)

                def idx_body(g, _):
                    o = g * 16
                    xg = xv[pl.ds(o, 16)]
                    yg = yv[pl.ds(o, 16)]
                    zg = zv[pl.ds(o, 16)]
                    px = xg * res
                    py = yg * res
                    pz = zg * res
                    ix = px.astype(jnp.int32)
                    iy = py.astype(jnp.int32)
                    iz = pz.astype(jnp.int32)
                    fx[pl.ds(o, 16)] = px - ix.astype(jnp.float32)
                    fy[pl.ds(o, 16)] = py - iy.astype(jnp.float32)
                    fz[pl.ds(o, 16)] = pz - iz.astype(jnp.float32)
                    hx0 = ix
                    hx1 = ix + 1
                    hy0 = iy * P1
                    hy1 = hy0 + P1
                    hz0 = iz * P2
                    hz1 = hz0 + P2
                    for c in range(8):
                        hx = hx1 if (c & 1) else hx0
                        hy = hy1 if (c & 2) else hy0
                        hz = hz1 if (c & 4) else hz0
                        e = ((hx ^ hy ^ hz) & IMASK) * 2 + lbase2
                        idxb[pl.ds(2 * c * C + o, 16)] = e
                        idxb[pl.ds((2 * c + 1) * C + o, 16)] = e + 1
                    return 0

                lax.fori_loop(0, NG, idx_body, 0)

                cps = [
                    pltpu.async_copy(
                        tab_h.at[idxb.at[pl.ds(j * 512, 512)]],
                        rows.at[pl.ds(j * 512, 512)], sem)
                    for j in range(16 * C // 512)
                ]
                for cp in cps:
                    cp.wait()

                def acc_body(g, _):
                    o = g * 16
                    fxg = fx[pl.ds(o, 16)]
                    fyg = fy[pl.ds(o, 16)]
                    fzg = fz[pl.ds(o, 16)]
                    gx = 1.0 - fxg
                    gy = 1.0 - fyg
                    gz = 1.0 - fzg
                    h0 = jnp.zeros((16,), jnp.float32)
                    h1 = jnp.zeros((16,), jnp.float32)
                    for c in range(8):
                        wx = fxg if (c & 1) else gx
                        wy = fyg if (c & 2) else gy
                        wz = fzg if (c & 4) else gz
                        wgt = wx * wy * wz
                        f0 = rows[pl.ds(2 * c * C + o, 16)]
                        f1 = rows[pl.ds((2 * c + 1) * C + o, 16)]
                        h0 = h0 + wgt * f0
                        h1 = h1 + wgt * f1
                    hbuf[2 * l, pl.ds(o, 16)] = h0
                    hbuf[2 * l + 1, pl.ds(o, 16)] = h1
                    return 0

                lax.fori_loop(0, NG, acc_body, 0)
                return 0

            lax.fori_loop(0, L, level_body, 0)
            ocps = [
                pltpu.async_copy(hbuf.at[ff],
                                 out_h.at[pl.ds(ff * N + base, C)], osem)
                for ff in range(2 * L)
            ]
            for cp in ocps:
                cp.wait()
            return 0

        lax.fori_loop(0, NCH, chunk_body, 0)

    return k(xs, ys, zs, tab1, resf)


BS = 2048
SH0 = 0.28209479177387814
SH1 = 0.48860251190291987
SH2 = 1.0925484305920792
SH3 = 0.94617469575755997
SH4 = 0.31539156525252005
SH5 = 0.54627421529603959
SH6 = 0.59004358992664352
SH7 = 2.8906114426405538
SH8 = 0.45704579946446572
SH9 = 0.3731763325901154
SH10 = 1.4453057213202769


def _scan_sum(x):
    d = 1
    n = x.shape[1]
    while d < n:
        x = x + jnp.concatenate([jnp.zeros((1, d), x.dtype), x[:, :-d]], axis=1)
        d *= 2
    return x


def _scan_max(x):
    d = 1
    n = x.shape[1]
    while d < n:
        x = jnp.maximum(
            x, jnp.concatenate([jnp.full((1, d), NEG, x.dtype), x[:, :-d]], axis=1))
        d *= 2
    return x


def _mlp_composite_tc(ht, dirs_t, deltas_t, seg2, segp2,
                      w1, b1r, w2, b2c, wr1, br1c, wr2, br2c, wr3, br3c):
    grid = (N // BS,)

    def body(ht_ref, d_ref, de_ref, s_ref, sp_ref,
             w1_ref, b1_ref, w2_ref, b2_ref, wr1_ref, br1_ref,
             wr2_ref, br2_ref, wr3_ref, br3_ref,
             o0_ref, o1_ref, o2_ref, o3_ref, carry):
        pid = pl.program_id(0)

        @pl.when(pid == 0)
        def _():
            carry[0] = 0.0
            carry[1] = NEG

        hts = ht_ref[...]
        hid = jax.nn.relu(
            lax.dot_general(hts, w1_ref[...], (((0,), (0,)), ((), ())),
                            preferred_element_type=jnp.float32) + b1_ref[...])
        g_t = lax.dot_general(w2_ref[...], hid, (((0,), (1,)), ((), ())),
                              preferred_element_type=jnp.float32) + b2_ref[...]
        sigma = jnp.exp(g_t[0:1, :])
        geo = g_t[1:17, :]

        d = d_ref[...]
        nrm = jnp.sqrt(jnp.sum(d * d, axis=0, keepdims=True)) + 1e-8
        dn = d / nrm
        x = dn[0:1, :]
        y = dn[1:2, :]
        z = dn[2:3, :]
        xx = x * x
        yy = y * y
        zz = z * z
        xy = x * y
        yz = y * z
        xz = x * z
        de = jnp.concatenate([
            jnp.full_like(x, SH0),
            -SH1 * y,
            SH1 * z,
            -SH1 * x,
            SH2 * xy,
            -SH2 * yz,
            SH3 * zz - SH4,
            -SH2 * xz,
            SH5 * (xx - yy),
            SH6 * y * (-3.0 * xx + yy),
            SH7 * xy * z,
            SH8 * y * (1.0 - 5.0 * zz),
            SH9 * z * (5.0 * zz - 3.0),
            SH8 * x * (1.0 - 5.0 * zz),
            SH10 * z * (xx - yy),
            SH6 * x * (-xx + 3.0 * yy),
        ], axis=0)
        ri = jnp.concatenate([de, geo], axis=0)
        h2 = jax.nn.relu(
            lax.dot_general(wr1_ref[...], ri, (((0,), (0,)), ((), ())),
                            preferred_element_type=jnp.float32) + br1_ref[...])
        h2 = jax.nn.relu(
            lax.dot_general(wr2_ref[...], h2, (((0,), (0,)), ((), ())),
                            preferred_element_type=jnp.float32) + br2_ref[...])
        rgb = jax.nn.sigmoid(
            lax.dot_general(wr3_ref[...], h2, (((0,), (0,)), ((), ())),
                            preferred_element_type=jnp.float32) + br3_ref[...])

        dlt = de_ref[...]
        dt = dlt[0:1, :] * 0.01
        ts = dlt[1:2, :]
        s = sigma * dt
        c0 = carry[0]
        c1 = carry[1]
        cs = _scan_sum(s)
        excl = (c0 + cs) - s
        bnd = s_ref[...] != sp_ref[...]
        cand = jnp.where(bnd, excl, NEG)
        off = jnp.maximum(_scan_max(cand), c1)
        trans = jnp.exp(-(excl - off))
        alpha = 1.0 - jnp.exp(-s)
        w = alpha * trans
        wrgb = w * rgb
        o0_ref[...] = wrgb[0]
        o1_ref[...] = wrgb[1]
        o2_ref[...] = wrgb[2]
        o3_ref[...] = (w * ts)[0]
        carry[0] = c0 + jnp.sum(s)
        carry[1] = jnp.maximum(jnp.max(cand), c1)

    full = lambda shape: pl.BlockSpec(shape, lambda i: (0, 0))
    blk = lambda r: pl.BlockSpec((r, BS), lambda i: (0, i))
    oblk = pl.BlockSpec((BS,), lambda i: (i,))
    o1d = jax.ShapeDtypeStruct((N,), jnp.float32)
    return pl.pallas_call(
        body,
        grid=grid,
        in_specs=[
            blk(2 * L), blk(3), blk(2), blk(1), blk(1),
            full((2 * L, 64)), full((1, 64)), full((64, 17)), full((17, 1)),
            full((32, 64)), full((64, 1)), full((64, 64)), full((64, 1)),
            full((64, 3)), full((3, 1)),
        ],
        out_specs=[oblk, oblk, oblk, oblk],
        out_shape=[o1d, o1d, o1d, o1d],
        scratch_shapes=[pltpu.SMEM((2,), jnp.float32)],
        compiler_params=pltpu.CompilerParams(
            dimension_semantics=("arbitrary",)),
    )(ht, dirs_t, deltas_t, seg2, segp2,
      w1, b1r, w2, b2c, wr1, br1c, wr2, br2c, wr3, br3c)


CC = 1024
NACC = 4 * N_RAYS


def _segsum_sc(v0a, v1a, v2a, v3a, seg, z1):
    mesh = plsc.VectorSubcoreMesh(core_axis_name="c", subcore_axis_name="s")

    @functools.partial(
        pl.kernel, mesh=mesh,
        out_type=jax.ShapeDtypeStruct((NW * NACC,), jnp.float32),
        scratch_types=[
            pltpu.VMEM((CC,), jnp.int32),
            pltpu.VMEM((CC,), jnp.float32),
            pltpu.VMEM((CC,), jnp.float32),
            pltpu.VMEM((CC,), jnp.float32),
            pltpu.VMEM((CC,), jnp.float32),
            pltpu.VMEM((NACC,), jnp.float32),
        ],
        compiler_params=pltpu.CompilerParams(needs_layout_passes=False),
    )
    def k(v0_h, v1_h, v2_h, v3_h, seg_h, z_h, out_h,
          segv, v0, v1, v2, v3, acc):
        wid = lax.axis_index("s") * 2 + lax.axis_index("c")
        tbase = wid * NT
        pltpu.sync_copy(z_h, acc)

        def chunk_body(ci, _):
            base = tbase + ci * CC
            pltpu.sync_copy(seg_h.at[pl.ds(base, CC)], segv)
            pltpu.sync_copy(v0_h.at[pl.ds(base, CC)], v0)
            pltpu.sync_copy(v1_h.at[pl.ds(base, CC)], v1)
            pltpu.sync_copy(v2_h.at[pl.ds(base, CC)], v2)
            pltpu.sync_copy(v3_h.at[pl.ds(base, CC)], v3)

            def g_body(g, _):
                o = g * 16
                sv = segv[pl.ds(o, 16)]
                plsc.addupdate_scatter(acc, [sv], v0[pl.ds(o, 16)])
                plsc.addupdate_scatter(acc, [sv + N_RAYS], v1[pl.ds(o, 16)])
                plsc.addupdate_scatter(acc, [sv + 2 * N_RAYS], v2[pl.ds(o, 16)])
                plsc.addupdate_scatter(acc, [sv + 3 * N_RAYS], v3[pl.ds(o, 16)])
                return 0

            lax.fori_loop(0, CC // 16, g_body, 0)
            return 0

        lax.fori_loop(0, NT // CC, chunk_body, 0)
        pltpu.sync_copy(acc, out_h.at[pl.ds(wid * NACC, NACC)])

    return k(v0a, v1a, v2a, v3a, seg, z1)


def _reduce_tc(partials):
    def body(p_ref, o_ref):
        o_ref[...] = jnp.sum(p_ref[...], axis=0)

    return pl.pallas_call(
        body,
        out_shape=jax.ShapeDtypeStruct((NACC,), jnp.float32),
    )(partials)


def kernel(xyzs, dirs, deltas, table, w1, b1, w2, b2, wr1, br1, wr2, br2,
           wr3, br3, segment_ids):
    xt = xyzs.T
    xs, ys, zs = xt[0], xt[1], xt[2]
    tab1 = table.reshape(L * T * 2)
    resf = jnp.asarray(RES, dtype=jnp.float32)

    ht1 = _hash_encode_sc(xs, ys, zs, tab1, resf)
    ht = ht1.reshape(2 * L, N)

    dirs_t = dirs.T
    deltas_t = deltas.T
    seg2 = segment_ids.reshape(1, N)
    segp2 = jnp.concatenate(
        [jnp.full((1,), -1, jnp.int32), segment_ids[:-1]]).reshape(1, N)
    v0a, v1a, v2a, v3a = _mlp_composite_tc(
        ht, dirs_t, deltas_t, seg2, segp2,
        w1, b1.reshape(1, 64), w2, b2.reshape(17, 1),
        wr1, br1.reshape(64, 1), wr2, br2.reshape(64, 1),
        wr3, br3.reshape(3, 1))

    z1 = jnp.zeros((NACC,), jnp.float32)
    partials = _segsum_sc(v0a, v1a, v2a, v3a, segment_ids, z1)
    out4 = _reduce_tc(partials.reshape(NW, NACC)).reshape(4, N_RAYS)
    image = out4[0:3].T
    depth = out4[3]
    return image, depth


# speedup vs baseline: 3.9571x; 3.9571x over previous
import functools
import numpy as np
import jax
import jax.numpy as jnp
from jax import lax
from jax.experimental import pallas as pl
from jax.experimental.pallas import tpu as pltpu
from jax.experimental.pallas import tpu_sc as plsc

L = 16
T = 2 ** 19
N_RAYS = 4096
N = 262144
P1 = np.int32(np.uint32(2654435761).view(np.int32))
P2 = np.int32(805459861)
IMASK = np.int32(T - 1)
HMASK = np.int32(-128)
RES = [float(np.floor(16.0 * 1.3819 ** l)) for l in range(L)]

NW = 32
NT = N // NW
C = 256
NCH = NT // C
NG = C // 16
NEG = -3.4e38


def _hash_encode_sc(xs, ys, zs, tab1, resf):
    mesh = plsc.VectorSubcoreMesh(core_axis_name="c", subcore_axis_name="s")

    @functools.partial(
        pl.kernel, mesh=mesh,
        out_type=jax.ShapeDtypeStruct((2 * L * N,), jnp.float32),
        scratch_types=[
            pltpu.VMEM((C,), jnp.float32),
            pltpu.VMEM((C,), jnp.float32),
            pltpu.VMEM((C,), jnp.float32),
            pltpu.VMEM((C,), jnp.float32),
            pltpu.VMEM((C,), jnp.float32),
            pltpu.VMEM((C,), jnp.float32),
            pltpu.VMEM((16 * C,), jnp.int32),
            pltpu.VMEM((16 * C,), jnp.float32),
            pltpu.VMEM((2 * L, C), jnp.float32),
            pltpu.VMEM((L,), jnp.float32),
            pltpu.SemaphoreType.DMA,
            pltpu.SemaphoreType.DMA,
        ],
        compiler_params=pltpu.CompilerParams(needs_layout_passes=False),
    )
    def k(xs_h, ys_h, zs_h, tab_h, res_h, out_h,
          xv, yv, zv, fx, fy, fz, idxb, rows, hbuf, resv, sem, osem):
        wid = lax.axis_index("s") * 2 + lax.axis_index("c")
        tbase = wid * NT
        pltpu.sync_copy(res_h, resv)

        def chunk_body(ci, _):
            base = tbase + ci * C
            pltpu.sync_copy(xs_h.at[pl.ds(base, C)], xv)
            pltpu.sync_copy(ys_h.at[pl.ds(base, C)], yv)
            pltpu.sync_copy(zs_h.at[pl.ds(base, C)], zv)

            def level_body(l, _):
                res = plsc.load_gather(resv, [jnp.full((16,), l, jnp.int32)])
                lbase2 = l * (2 * T)

                def idx_body(g, _):
                    o = g * 16
                    xg = xv[pl.ds(o, 16)]
                    yg = yv[pl.ds(o, 16)]
                    zg = zv[pl.ds(o, 16)]
                    px = xg * res
                    py = yg * res
                    pz = zg * res
                    ix = px.astype(jnp.int32)
                    iy = py.astype(jnp.int32)
                    iz = pz.astype(jnp.int32)
                    fx[pl.ds(o, 16)] = px - ix.astype(jnp.float32)
                    fy[pl.ds(o, 16)] = py - iy.astype(jnp.float32)
                    fz[pl.ds(o, 16)] = pz - iz.astype(jnp.float32)
                    hx0 = ix
                    hx1 = ix + 1
                    hy0 = iy * P1
                    hy1 = hy0 + P1
                    hz0 = iz * P2
                    hz1 = hz0 + P2
                    for c in range(8):
                        hx = hx1 if (c & 1) else hx0
                        hy = hy1 if (c & 2) else hy0
                        hz = hz1 if (c & 4) else hz0
                        t = (hx ^ hy ^ hz) & IMASK
                        e = t + (t & HMASK) + lbase2
                        idxb[pl.ds(2 * c * C + o, 16)] = e
                        idxb[pl.ds((2 * c + 1) * C + o, 16)] = e + 128
                    return 0

                lax.fori_loop(0, NG, idx_body, 0)

                cps = [
                    pltpu.async_copy(
                        tab_h.at[idxb.at[pl.ds(j * 512, 512)]],
                        rows.at[pl.ds(j * 512, 512)], sem)
                    for j in range(16 * C // 512)
                ]
                for cp in cps:
                    cp.wait()

                def acc_body(g, _):
                    o = g * 16
                    fxg = fx[pl.ds(o, 16)]
                    fyg = fy[pl.ds(o, 16)]
                    fzg = fz[pl.ds(o, 16)]
                    gx = 1.0 - fxg
                    gy = 1.0 - fyg
                    gz = 1.0 - fzg
                    h0 = jnp.zeros((16,), jnp.float32)
                    h1 = jnp.zeros((16,), jnp.float32)
                    for c in range(8):
                        wx = fxg if (c & 1) else gx
                        wy = fyg if (c & 2) else gy
                        wz = fzg if (c & 4) else gz
                        wgt = wx * wy * wz
                        f0 = rows[pl.ds(2 * c * C + o, 16)]
                        f1 = rows[pl.ds((2 * c + 1) * C + o, 16)]
                        h0 = h0 + wgt * f0
                        h1 = h1 + wgt * f1
                    hbuf[2 * l, pl.ds(o, 16)] = h0
                    hbuf[2 * l + 1, pl.ds(o, 16)] = h1
                    return 0

                lax.fori_loop(0, NG, acc_body, 0)
                return 0

            lax.fori_loop(0, L, level_body, 0)
            ocps = [
                pltpu.async_copy(hbuf.at[ff],
                                 out_h.at[pl.ds(ff * N + base, C)], osem)
                for ff in range(2 * L)
            ]
            for cp in ocps:
                cp.wait()
            return 0

        lax.fori_loop(0, NCH, chunk_body, 0)

    return k(xs, ys, zs, tab1, resf)


BS = 2048
SH0 = 0.28209479177387814
SH1 = 0.48860251190291987
SH2 = 1.0925484305920792
SH3 = 0.94617469575755997
SH4 = 0.31539156525252005
SH5 = 0.54627421529603959
SH6 = 0.59004358992664352
SH7 = 2.8906114426405538
SH8 = 0.45704579946446572
SH9 = 0.3731763325901154
SH10 = 1.4453057213202769


def _scan_sum(x):
    d = 1
    n = x.shape[1]
    while d < n:
        x = x + jnp.concatenate([jnp.zeros((1, d), x.dtype), x[:, :-d]], axis=1)
        d *= 2
    return x


def _scan_max(x):
    d = 1
    n = x.shape[1]
    while d < n:
        x = jnp.maximum(
            x, jnp.concatenate([jnp.full((1, d), NEG, x.dtype), x[:, :-d]], axis=1))
        d *= 2
    return x


def _mlp_composite_tc(ht, dirs_t, deltas_t, seg2, segp2,
                      w1, b1r, w2, b2c, wr1, br1c, wr2, br2c, wr3, br3c):
    grid = (N // BS,)

    def body(ht_ref, d_ref, de_ref, s_ref, sp_ref,
             w1_ref, b1_ref, w2_ref, b2_ref, wr1_ref, br1_ref,
             wr2_ref, br2_ref, wr3_ref, br3_ref,
             o0_ref, o1_ref, o2_ref, o3_ref, carry):
        pid = pl.program_id(0)

        @pl.when(pid == 0)
        def _():
            carry[0] = 0.0
            carry[1] = NEG

        hts = ht_ref[...]
        hid = jax.nn.relu(
            lax.dot_general(hts, w1_ref[...], (((0,), (0,)), ((), ())),
                            preferred_element_type=jnp.float32) + b1_ref[...])
        g_t = lax.dot_general(w2_ref[...], hid, (((0,), (1,)), ((), ())),
                              preferred_element_type=jnp.float32) + b2_ref[...]
        sigma = jnp.exp(g_t[0:1, :])
        geo = g_t[1:17, :]

        d = d_ref[...]
        nrm = jnp.sqrt(jnp.sum(d * d, axis=0, keepdims=True)) + 1e-8
        dn = d / nrm
        x = dn[0:1, :]
        y = dn[1:2, :]
        z = dn[2:3, :]
        xx = x * x
        yy = y * y
        zz = z * z
        xy = x * y
        yz = y * z
        xz = x * z
        de = jnp.concatenate([
            jnp.full_like(x, SH0),
            -SH1 * y,
            SH1 * z,
            -SH1 * x,
            SH2 * xy,
            -SH2 * yz,
            SH3 * zz - SH4,
            -SH2 * xz,
            SH5 * (xx - yy),
            SH6 * y * (-3.0 * xx + yy),
            SH7 * xy * z,
            SH8 * y * (1.0 - 5.0 * zz),
            SH9 * z * (5.0 * zz - 3.0),
            SH8 * x * (1.0 - 5.0 * zz),
            SH10 * z * (xx - yy),
            SH6 * x * (-xx + 3.0 * yy),
        ], axis=0)
        ri = jnp.concatenate([de, geo], axis=0)
        h2 = jax.nn.relu(
            lax.dot_general(wr1_ref[...], ri, (((0,), (0,)), ((), ())),
                            preferred_element_type=jnp.float32) + br1_ref[...])
        h2 = jax.nn.relu(
            lax.dot_general(wr2_ref[...], h2, (((0,), (0,)), ((), ())),
                            preferred_element_type=jnp.float32) + br2_ref[...])
        rgb = jax.nn.sigmoid(
            lax.dot_general(wr3_ref[...], h2, (((0,), (0,)), ((), ())),
                            preferred_element_type=jnp.float32) + br3_ref[...])

        dlt = de_ref[...]
        dt = dlt[0:1, :] * 0.01
        ts = dlt[1:2, :]
        s = sigma * dt
        c0 = carry[0]
        c1 = carry[1]
        cs = _scan_sum(s)
        excl = (c0 + cs) - s
        bnd = s_ref[...] != sp_ref[...]
        cand = jnp.where(bnd, excl, NEG)
        off = jnp.maximum(_scan_max(cand), c1)
        trans = jnp.exp(-(excl - off))
        alpha = 1.0 - jnp.exp(-s)
        w = alpha * trans
        wrgb = w * rgb
        o0_ref[...] = wrgb[0]
        o1_ref[...] = wrgb[1]
        o2_ref[...] = wrgb[2]
        o3_ref[...] = (w * ts)[0]
        carry[0] = c0 + jnp.sum(s)
        carry[1] = jnp.maximum(jnp.max(cand), c1)

    full = lambda shape: pl.BlockSpec(shape, lambda i: (0, 0))
    blk = lambda r: pl.BlockSpec((r, BS), lambda i: (0, i))
    oblk = pl.BlockSpec((BS,), lambda i: (i,))
    o1d = jax.ShapeDtypeStruct((N,), jnp.float32)
    return pl.pallas_call(
        body,
        grid=grid,
        in_specs=[
            blk(2 * L), blk(3), blk(2), blk(1), blk(1),
            full((2 * L, 64)), full((1, 64)), full((64, 17)), full((17, 1)),
            full((32, 64)), full((64, 1)), full((64, 64)), full((64, 1)),
            full((64, 3)), full((3, 1)),
        ],
        out_specs=[oblk, oblk, oblk, oblk],
        out_shape=[o1d, o1d, o1d, o1d],
        scratch_shapes=[pltpu.SMEM((2,), jnp.float32)],
        compiler_params=pltpu.CompilerParams(
            dimension_semantics=("arbitrary",)),
    )(ht, dirs_t, deltas_t, seg2, segp2,
      w1, b1r, w2, b2c, wr1, br1c, wr2, br2c, wr3, br3c)


CC = 1024
NACC = 4 * N_RAYS


def _segsum_sc(v0a, v1a, v2a, v3a, seg, z1):
    mesh = plsc.VectorSubcoreMesh(core_axis_name="c", subcore_axis_name="s")

    @functools.partial(
        pl.kernel, mesh=mesh,
        out_type=jax.ShapeDtypeStruct((NW * NACC,), jnp.float32),
        scratch_types=[
            pltpu.VMEM((CC,), jnp.int32),
            pltpu.VMEM((CC,), jnp.float32),
            pltpu.VMEM((CC,), jnp.float32),
            pltpu.VMEM((CC,), jnp.float32),
            pltpu.VMEM((CC,), jnp.float32),
            pltpu.VMEM((NACC,), jnp.float32),
        ],
        compiler_params=pltpu.CompilerParams(needs_layout_passes=False),
    )
    def k(v0_h, v1_h, v2_h, v3_h, seg_h, z_h, out_h,
          segv, v0, v1, v2, v3, acc):
        wid = lax.axis_index("s") * 2 + lax.axis_index("c")
        tbase = wid * NT
        pltpu.sync_copy(z_h, acc)

        def chunk_body(ci, _):
            base = tbase + ci * CC
            pltpu.sync_copy(seg_h.at[pl.ds(base, CC)], segv)
            pltpu.sync_copy(v0_h.at[pl.ds(base, CC)], v0)
            pltpu.sync_copy(v1_h.at[pl.ds(base, CC)], v1)
            pltpu.sync_copy(v2_h.at[pl.ds(base, CC)], v2)
            pltpu.sync_copy(v3_h.at[pl.ds(base, CC)], v3)

            def g_body(g, _):
                o = g * 16
                sv = segv[pl.ds(o, 16)]
                plsc.addupdate_scatter(acc, [sv], v0[pl.ds(o, 16)])
                plsc.addupdate_scatter(acc, [sv + N_RAYS], v1[pl.ds(o, 16)])
                plsc.addupdate_scatter(acc, [sv + 2 * N_RAYS], v2[pl.ds(o, 16)])
                plsc.addupdate_scatter(acc, [sv + 3 * N_RAYS], v3[pl.ds(o, 16)])
                return 0

            lax.fori_loop(0, CC // 16, g_body, 0)
            return 0

        lax.fori_loop(0, NT // CC, chunk_body, 0)
        pltpu.sync_copy(acc, out_h.at[pl.ds(wid * NACC, NACC)])

    return k(v0a, v1a, v2a, v3a, seg, z1)


def _reduce_tc(partials):
    def body(p_ref, o_ref):
        o_ref[...] = jnp.sum(p_ref[...], axis=0)

    return pl.pallas_call(
        body,
        out_shape=jax.ShapeDtypeStruct((NACC,), jnp.float32),
    )(partials)


def kernel(xyzs, dirs, deltas, table, w1, b1, w2, b2, wr1, br1, wr2, br2,
           wr3, br3, segment_ids):
    xt = xyzs.T
    xs, ys, zs = xt[0], xt[1], xt[2]
    tab1 = table.reshape(L, T // 128, 128, 2).transpose(0, 1, 3, 2).reshape(
        L * T * 2)
    resf = jnp.asarray(RES, dtype=jnp.float32)

    ht1 = _hash_encode_sc(xs, ys, zs, tab1, resf)
    ht = ht1.reshape(2 * L, N)

    dirs_t = dirs.T
    deltas_t = deltas.T
    seg2 = segment_ids.reshape(1, N)
    segp2 = jnp.concatenate(
        [jnp.full((1,), -1, jnp.int32), segment_ids[:-1]]).reshape(1, N)
    v0a, v1a, v2a, v3a = _mlp_composite_tc(
        ht, dirs_t, deltas_t, seg2, segp2,
        w1, b1.reshape(1, 64), w2, b2.reshape(17, 1),
        wr1, br1.reshape(64, 1), wr2, br2.reshape(64, 1),
        wr3, br3.reshape(3, 1))

    z1 = jnp.zeros((NACC,), jnp.float32)
    partials = _segsum_sc(v0a, v1a, v2a, v3a, segment_ids, z1)
    out4 = _reduce_tc(partials.reshape(NW, NACC)).reshape(4, N_RAYS)
    image = out4[0:3].T
    depth = out4[3]
    return image, depth


# speedup vs baseline: 3.9606x; 1.0009x over previous
import functools
import numpy as np
import jax
import jax.numpy as jnp
from jax import lax
from jax.experimental import pallas as pl
from jax.experimental.pallas import tpu as pltpu
from jax.experimental.pallas import tpu_sc as plsc

L = 16
T = 2 ** 19
N_RAYS = 4096
N = 262144
P1 = np.int32(np.uint32(2654435761).view(np.int32))
P2 = np.int32(805459861)
IMASK = np.int32(T - 1)
HMASK = np.int32(-128)
RES = [float(np.floor(16.0 * 1.3819 ** l)) for l in range(L)]

NW = 32
NT = N // NW
C = 256
NCH = NT // C
NG = C // 16
NEG = -3.4e38


def _hash_encode_sc(xs, ys, zs, tab1, resf):
    mesh = plsc.VectorSubcoreMesh(core_axis_name="c", subcore_axis_name="s")

    @functools.partial(
        pl.kernel, mesh=mesh,
        out_type=jax.ShapeDtypeStruct((2 * L * N,), jnp.float32),
        scratch_types=[
            pltpu.VMEM((C,), jnp.float32),
            pltpu.VMEM((C,), jnp.float32),
            pltpu.VMEM((C,), jnp.float32),
            pltpu.VMEM((C,), jnp.float32),
            pltpu.VMEM((C,), jnp.float32),
            pltpu.VMEM((C,), jnp.float32),
            pltpu.VMEM((16 * C,), jnp.int32),
            pltpu.VMEM((16 * C,), jnp.float32),
            pltpu.VMEM((2 * L, C), jnp.float32),
            pltpu.VMEM((L,), jnp.float32),
            pltpu.SemaphoreType.DMA,
            pltpu.SemaphoreType.DMA,
        ],
        compiler_params=pltpu.CompilerParams(needs_layout_passes=False),
    )
    def k(xs_h, ys_h, zs_h, tab_h, res_h, out_h,
          xv, yv, zv, fx, fy, fz, idxb, rows, hbuf, resv, sem, osem):
        wid = lax.axis_index("s") * 2 + lax.axis_index("c")
        tbase = wid * NT
        pltpu.sync_copy(res_h, resv)

        def chunk_body(ci, _):
            base = tbase + ci * C
            pltpu.sync_copy(xs_h.at[pl.ds(base, C)], xv)
            pltpu.sync_copy(ys_h.at[pl.ds(base, C)], yv)
            pltpu.sync_copy(zs_h.at[pl.ds(base, C)], zv)

            def level_body(l, _):
                res = plsc.load_gather(resv, [jnp.full((16,), l, jnp.int32)])
                lbase2 = l * (2 * T)

                def idx_body(g, _):
                    o = g * 16
                    xg = xv[pl.ds(o, 16)]
                    yg = yv[pl.ds(o, 16)]
                    zg = zv[pl.ds(o, 16)]
                    px = xg * res
                    py = yg * res
                    pz = zg * res
                    ix = px.astype(jnp.int32)
                    iy = py.astype(jnp.int32)
                    iz = pz.astype(jnp.int32)
                    fx[pl.ds(o, 16)] = px - ix.astype(jnp.float32)
                    fy[pl.ds(o, 16)] = py - iy.astype(jnp.float32)
                    fz[pl.ds(o, 16)] = pz - iz.astype(jnp.float32)
                    hx0 = ix
                    hx1 = ix + 1
                    hy0 = iy * P1
                    hy1 = hy0 + P1
                    hz0 = iz * P2
                    hz1 = hz0 + P2
                    for c in range(8):
                        hx = hx1 if (c & 1) else hx0
                        hy = hy1 if (c & 2) else hy0
                        hz = hz1 if (c & 4) else hz0
                        t = (hx ^ hy ^ hz) & IMASK
                        e = t + (t & HMASK) + lbase2
                        idxb[pl.ds(2 * c * C + o, 16)] = e
                        idxb[pl.ds((2 * c + 1) * C + o, 16)] = e + 128
                    return 0

                lax.fori_loop(0, NG, idx_body, 0)

                pltpu.async_copy(tab_h.at[idxb], rows, sem).wait()

                def acc_body(g, _):
                    o = g * 16
                    fxg = fx[pl.ds(o, 16)]
                    fyg = fy[pl.ds(o, 16)]
                    fzg = fz[pl.ds(o, 16)]
                    gx = 1.0 - fxg
                    gy = 1.0 - fyg
                    gz = 1.0 - fzg
                    h0 = jnp.zeros((16,), jnp.float32)
                    h1 = jnp.zeros((16,), jnp.float32)
                    for c in range(8):
                        wx = fxg if (c & 1) else gx
                        wy = fyg if (c & 2) else gy
                        wz = fzg if (c & 4) else gz
                        wgt = wx * wy * wz
                        f0 = rows[pl.ds(2 * c * C + o, 16)]
                        f1 = rows[pl.ds((2 * c + 1) * C + o, 16)]
                        h0 = h0 + wgt * f0
                        h1 = h1 + wgt * f1
                    hbuf[2 * l, pl.ds(o, 16)] = h0
                    hbuf[2 * l + 1, pl.ds(o, 16)] = h1
                    return 0

                lax.fori_loop(0, NG, acc_body, 0)
                return 0

            lax.fori_loop(0, L, level_body, 0)
            ocps = [
                pltpu.async_copy(hbuf.at[ff],
                                 out_h.at[pl.ds(ff * N + base, C)], osem)
                for ff in range(2 * L)
            ]
            for cp in ocps:
                cp.wait()
            return 0

        lax.fori_loop(0, NCH, chunk_body, 0)

    return k(xs, ys, zs, tab1, resf)


BS = 2048
SH0 = 0.28209479177387814
SH1 = 0.48860251190291987
SH2 = 1.0925484305920792
SH3 = 0.94617469575755997
SH4 = 0.31539156525252005
SH5 = 0.54627421529603959
SH6 = 0.59004358992664352
SH7 = 2.8906114426405538
SH8 = 0.45704579946446572
SH9 = 0.3731763325901154
SH10 = 1.4453057213202769


def _scan_sum(x):
    d = 1
    n = x.shape[1]
    while d < n:
        x = x + jnp.concatenate([jnp.zeros((1, d), x.dtype), x[:, :-d]], axis=1)
        d *= 2
    return x


def _scan_max(x):
    d = 1
    n = x.shape[1]
    while d < n:
        x = jnp.maximum(
            x, jnp.concatenate([jnp.full((1, d), NEG, x.dtype), x[:, :-d]], axis=1))
        d *= 2
    return x


def _mlp_composite_tc(ht, dirs_t, deltas_t, seg2, segp2,
                      w1, b1r, w2, b2c, wr1, br1c, wr2, br2c, wr3, br3c):
    grid = (N // BS,)

    def body(ht_ref, d_ref, de_ref, s_ref, sp_ref,
             w1_ref, b1_ref, w2_ref, b2_ref, wr1_ref, br1_ref,
             wr2_ref, br2_ref, wr3_ref, br3_ref,
             o0_ref, o1_ref, o2_ref, o3_ref, carry):
        pid = pl.program_id(0)

        @pl.when(pid == 0)
        def _():
            carry[0] = 0.0
            carry[1] = NEG

        hts = ht_ref[...]
        hid = jax.nn.relu(
            lax.dot_general(hts, w1_ref[...], (((0,), (0,)), ((), ())),
                            preferred_element_type=jnp.float32) + b1_ref[...])
        g_t = lax.dot_general(w2_ref[...], hid, (((0,), (1,)), ((), ())),
                              preferred_element_type=jnp.float32) + b2_ref[...]
        sigma = jnp.exp(g_t[0:1, :])
        geo = g_t[1:17, :]

        d = d_ref[...]
        nrm = jnp.sqrt(jnp.sum(d * d, axis=0, keepdims=True)) + 1e-8
        dn = d / nrm
        x = dn[0:1, :]
        y = dn[1:2, :]
        z = dn[2:3, :]
        xx = x * x
        yy = y * y
        zz = z * z
        xy = x * y
        yz = y * z
        xz = x * z
        de = jnp.concatenate([
            jnp.full_like(x, SH0),
            -SH1 * y,
            SH1 * z,
            -SH1 * x,
            SH2 * xy,
            -SH2 * yz,
            SH3 * zz - SH4,
            -SH2 * xz,
            SH5 * (xx - yy),
            SH6 * y * (-3.0 * xx + yy),
            SH7 * xy * z,
            SH8 * y * (1.0 - 5.0 * zz),
            SH9 * z * (5.0 * zz - 3.0),
            SH8 * x * (1.0 - 5.0 * zz),
            SH10 * z * (xx - yy),
            SH6 * x * (-xx + 3.0 * yy),
        ], axis=0)
        ri = jnp.concatenate([de, geo], axis=0)
        h2 = jax.nn.relu(
            lax.dot_general(wr1_ref[...], ri, (((0,), (0,)), ((), ())),
                            preferred_element_type=jnp.float32) + br1_ref[...])
        h2 = jax.nn.relu(
            lax.dot_general(wr2_ref[...], h2, (((0,), (0,)), ((), ())),
                            preferred_element_type=jnp.float32) + br2_ref[...])
        rgb = jax.nn.sigmoid(
            lax.dot_general(wr3_ref[...], h2, (((0,), (0,)), ((), ())),
                            preferred_element_type=jnp.float32) + br3_ref[...])

        dlt = de_ref[...]
        dt = dlt[0:1, :] * 0.01
        ts = dlt[1:2, :]
        s = sigma * dt
        c0 = carry[0]
        c1 = carry[1]
        cs = _scan_sum(s)
        excl = (c0 + cs) - s
        bnd = s_ref[...] != sp_ref[...]
        cand = jnp.where(bnd, excl, NEG)
        off = jnp.maximum(_scan_max(cand), c1)
        trans = jnp.exp(-(excl - off))
        alpha = 1.0 - jnp.exp(-s)
        w = alpha * trans
        wrgb = w * rgb
        o0_ref[...] = wrgb[0]
        o1_ref[...] = wrgb[1]
        o2_ref[...] = wrgb[2]
        o3_ref[...] = (w * ts)[0]
        carry[0] = c0 + jnp.sum(s)
        carry[1] = jnp.maximum(jnp.max(cand), c1)

    full = lambda shape: pl.BlockSpec(shape, lambda i: (0, 0))
    blk = lambda r: pl.BlockSpec((r, BS), lambda i: (0, i))
    oblk = pl.BlockSpec((BS,), lambda i: (i,))
    o1d = jax.ShapeDtypeStruct((N,), jnp.float32)
    return pl.pallas_call(
        body,
        grid=grid,
        in_specs=[
            blk(2 * L), blk(3), blk(2), blk(1), blk(1),
            full((2 * L, 64)), full((1, 64)), full((64, 17)), full((17, 1)),
            full((32, 64)), full((64, 1)), full((64, 64)), full((64, 1)),
            full((64, 3)), full((3, 1)),
        ],
        out_specs=[oblk, oblk, oblk, oblk],
        out_shape=[o1d, o1d, o1d, o1d],
        scratch_shapes=[pltpu.SMEM((2,), jnp.float32)],
        compiler_params=pltpu.CompilerParams(
            dimension_semantics=("arbitrary",)),
    )(ht, dirs_t, deltas_t, seg2, segp2,
      w1, b1r, w2, b2c, wr1, br1c, wr2, br2c, wr3, br3c)


CC = 1024
NACC = 4 * N_RAYS


def _segsum_sc(v0a, v1a, v2a, v3a, seg, z1):
    mesh = plsc.VectorSubcoreMesh(core_axis_name="c", subcore_axis_name="s")

    @functools.partial(
        pl.kernel, mesh=mesh,
        out_type=jax.ShapeDtypeStruct((NW * NACC,), jnp.float32),
        scratch_types=[
            pltpu.VMEM((CC,), jnp.int32),
            pltpu.VMEM((CC,), jnp.float32),
            pltpu.VMEM((CC,), jnp.float32),
            pltpu.VMEM((CC,), jnp.float32),
            pltpu.VMEM((CC,), jnp.float32),
            pltpu.VMEM((NACC,), jnp.float32),
        ],
        compiler_params=pltpu.CompilerParams(needs_layout_passes=False),
    )
    def k(v0_h, v1_h, v2_h, v3_h, seg_h, z_h, out_h,
          segv, v0, v1, v2, v3, acc):
        wid = lax.axis_index("s") * 2 + lax.axis_index("c")
        tbase = wid * NT
        pltpu.sync_copy(z_h, acc)

        def chunk_body(ci, _):
            base = tbase + ci * CC
            pltpu.sync_copy(seg_h.at[pl.ds(base, CC)], segv)
            pltpu.sync_copy(v0_h.at[pl.ds(base, CC)], v0)
            pltpu.sync_copy(v1_h.at[pl.ds(base, CC)], v1)
            pltpu.sync_copy(v2_h.at[pl.ds(base, CC)], v2)
            pltpu.sync_copy(v3_h.at[pl.ds(base, CC)], v3)

            def g_body(g, _):
                o = g * 16
                sv = segv[pl.ds(o, 16)]
                plsc.addupdate_scatter(acc, [sv], v0[pl.ds(o, 16)])
                plsc.addupdate_scatter(acc, [sv + N_RAYS], v1[pl.ds(o, 16)])
                plsc.addupdate_scatter(acc, [sv + 2 * N_RAYS], v2[pl.ds(o, 16)])
                plsc.addupdate_scatter(acc, [sv + 3 * N_RAYS], v3[pl.ds(o, 16)])
                return 0

            lax.fori_loop(0, CC // 16, g_body, 0)
            return 0

        lax.fori_loop(0, NT // CC, chunk_body, 0)
        pltpu.sync_copy(acc, out_h.at[pl.ds(wid * NACC, NACC)])

    return k(v0a, v1a, v2a, v3a, seg, z1)


def _reduce_tc(partials):
    def body(p_ref, o_ref):
        o_ref[...] = jnp.sum(p_ref[...], axis=0)

    return pl.pallas_call(
        body,
        out_shape=jax.ShapeDtypeStruct((NACC,), jnp.float32),
    )(partials)


def kernel(xyzs, dirs, deltas, table, w1, b1, w2, b2, wr1, br1, wr2, br2,
           wr3, br3, segment_ids):
    xt = xyzs.T
    xs, ys, zs = xt[0], xt[1], xt[2]
    tab1 = table.reshape(L, T // 128, 128, 2).transpose(0, 1, 3, 2).reshape(
        L * T * 2)
    resf = jnp.asarray(RES, dtype=jnp.float32)

    ht1 = _hash_encode_sc(xs, ys, zs, tab1, resf)
    ht = ht1.reshape(2 * L, N)

    dirs_t = dirs.T
    deltas_t = deltas.T
    seg2 = segment_ids.reshape(1, N)
    segp2 = jnp.concatenate(
        [jnp.full((1,), -1, jnp.int32), segment_ids[:-1]]).reshape(1, N)
    v0a, v1a, v2a, v3a = _mlp_composite_tc(
        ht, dirs_t, deltas_t, seg2, segp2,
        w1, b1.reshape(1, 64), w2, b2.reshape(17, 1),
        wr1, br1.reshape(64, 1), wr2, br2.reshape(64, 1),
        wr3, br3.reshape(3, 1))

    z1 = jnp.zeros((NACC,), jnp.float32)
    partials = _segsum_sc(v0a, v1a, v2a, v3a, segment_ids, z1)
    out4 = _reduce_tc(partials.reshape(NW, NACC)).reshape(4, N_RAYS)
    image = out4[0:3].T
    depth = out4[3]
    return image, depth


# speedup vs baseline: 4.4842x; 1.1322x over previous
import functools
import numpy as np
import jax
import jax.numpy as jnp
from jax import lax
from jax.experimental import pallas as pl
from jax.experimental.pallas import tpu as pltpu
from jax.experimental.pallas import tpu_sc as plsc

L = 16
T = 2 ** 19
N_RAYS = 4096
N = 262144
P1 = np.int32(np.uint32(2654435761).view(np.int32))
P2 = np.int32(805459861)
IMASK = np.int32(T - 1)
HMASK = np.int32(-128)
RES = [float(np.floor(16.0 * 1.3819 ** l)) for l in range(L)]

NW = 32
NT = N // NW
C = 256
NCH = NT // C
NG = C // 16
NEG = -3.4e38


def _hash_encode_sc(xs, ys, zs, tab1, resf):
    mesh = plsc.VectorSubcoreMesh(core_axis_name="c", subcore_axis_name="s")

    @functools.partial(
        pl.kernel, mesh=mesh,
        out_type=jax.ShapeDtypeStruct((2 * L * N,), jnp.float32),
        scratch_types=[
            pltpu.VMEM((C,), jnp.float32),
            pltpu.VMEM((C,), jnp.float32),
            pltpu.VMEM((C,), jnp.float32),
            pltpu.VMEM((C,), jnp.float32),
            pltpu.VMEM((C,), jnp.float32),
            pltpu.VMEM((C,), jnp.float32),
            pltpu.VMEM((C,), jnp.float32),
            pltpu.VMEM((C,), jnp.float32),
            pltpu.VMEM((C,), jnp.float32),
            pltpu.VMEM((16 * C,), jnp.int32),
            pltpu.VMEM((16 * C,), jnp.int32),
            pltpu.VMEM((16 * C,), jnp.float32),
            pltpu.VMEM((16 * C,), jnp.float32),
            pltpu.VMEM((2 * L, C), jnp.float32),
            pltpu.VMEM((L,), jnp.float32),
            pltpu.SemaphoreType.DMA,
            pltpu.SemaphoreType.DMA,
        ],
        compiler_params=pltpu.CompilerParams(needs_layout_passes=False),
    )
    def k(xs_h, ys_h, zs_h, tab_h, res_h, out_h,
          xv, yv, zv, fxa, fya, fza, fxb, fyb, fzb,
          idxa, idxb2, rowa, rowb, hbuf, resv, sem, osem):
        wid = lax.axis_index("s") * 2 + lax.axis_index("c")
        tbase = wid * NT
        pltpu.sync_copy(res_h, resv)

        def chunk_body(ci, _):
            base = tbase + ci * C
            pltpu.sync_copy(xs_h.at[pl.ds(base, C)], xv)
            pltpu.sync_copy(ys_h.at[pl.ds(base, C)], yv)
            pltpu.sync_copy(zs_h.at[pl.ds(base, C)], zv)

            bufA = (fxa, fya, fza, idxa, rowa)
            bufB = (fxb, fyb, fzb, idxb2, rowb)

            def compute_idx(lv, buf):
                fx, fy, fz, idxb, rows = buf
                res = plsc.load_gather(resv, [jnp.full((16,), lv, jnp.int32)])
                lbase2 = lv * (2 * T)

                def idx_body(g, _):
                    o = g * 16
                    xg = xv[pl.ds(o, 16)]
                    yg = yv[pl.ds(o, 16)]
                    zg = zv[pl.ds(o, 16)]
                    px = xg * res
                    py = yg * res
                    pz = zg * res
                    ix = px.astype(jnp.int32)
                    iy = py.astype(jnp.int32)
                    iz = pz.astype(jnp.int32)
                    fx[pl.ds(o, 16)] = px - ix.astype(jnp.float32)
                    fy[pl.ds(o, 16)] = py - iy.astype(jnp.float32)
                    fz[pl.ds(o, 16)] = pz - iz.astype(jnp.float32)
                    hx0 = ix
                    hx1 = ix + 1
                    hy0 = iy * P1
                    hy1 = hy0 + P1
                    hz0 = iz * P2
                    hz1 = hz0 + P2
                    for c in range(8):
                        hx = hx1 if (c & 1) else hx0
                        hy = hy1 if (c & 2) else hy0
                        hz = hz1 if (c & 4) else hz0
                        t = (hx ^ hy ^ hz) & IMASK
                        e = t + (t & HMASK) + lbase2
                        idxb[pl.ds(2 * c * C + o, 16)] = e
                        idxb[pl.ds((2 * c + 1) * C + o, 16)] = e + 128
                    return 0

                lax.fori_loop(0, NG, idx_body, 0)

            def issue(buf):
                return pltpu.async_copy(tab_h.at[buf[3]], buf[4], sem)

            def wait(buf):
                pltpu.make_async_copy(tab_h.at[buf[3]], buf[4], sem).wait()

            def accum(lv, buf):
                fx, fy, fz, idxb, rows = buf

                def acc_body(g, _):
                    o = g * 16
                    fxg = fx[pl.ds(o, 16)]
                    fyg = fy[pl.ds(o, 16)]
                    fzg = fz[pl.ds(o, 16)]
                    gx = 1.0 - fxg
                    gy = 1.0 - fyg
                    gz = 1.0 - fzg
                    h0 = jnp.zeros((16,), jnp.float32)
                    h1 = jnp.zeros((16,), jnp.float32)
                    for c in range(8):
                        wx = fxg if (c & 1) else gx
                        wy = fyg if (c & 2) else gy
                        wz = fzg if (c & 4) else gz
                        wgt = wx * wy * wz
                        f0 = rows[pl.ds(2 * c * C + o, 16)]
                        f1 = rows[pl.ds((2 * c + 1) * C + o, 16)]
                        h0 = h0 + wgt * f0
                        h1 = h1 + wgt * f1
                    hbuf[2 * lv, pl.ds(o, 16)] = h0
                    hbuf[2 * lv + 1, pl.ds(o, 16)] = h1
                    return 0

                lax.fori_loop(0, NG, acc_body, 0)

            compute_idx(0, bufA)
            issue(bufA)

            def level_body(i, _):
                lv = 2 * i
                compute_idx(lv + 1, bufB)
                issue(bufB)
                wait(bufA)
                accum(lv, bufA)

                @pl.when(i < L // 2 - 1)
                def _():
                    compute_idx(lv + 2, bufA)
                    issue(bufA)

                wait(bufB)
                accum(lv + 1, bufB)
                return 0

            lax.fori_loop(0, L // 2, level_body, 0)
            ocps = [
                pltpu.async_copy(hbuf.at[ff],
                                 out_h.at[pl.ds(ff * N + base, C)], osem)
                for ff in range(2 * L)
            ]
            for cp in ocps:
                cp.wait()
            return 0

        lax.fori_loop(0, NCH, chunk_body, 0)

    return k(xs, ys, zs, tab1, resf)


BS = 2048
SH0 = 0.28209479177387814
SH1 = 0.48860251190291987
SH2 = 1.0925484305920792
SH3 = 0.94617469575755997
SH4 = 0.31539156525252005
SH5 = 0.54627421529603959
SH6 = 0.59004358992664352
SH7 = 2.8906114426405538
SH8 = 0.45704579946446572
SH9 = 0.3731763325901154
SH10 = 1.4453057213202769


def _scan_sum(x):
    d = 1
    n = x.shape[1]
    while d < n:
        x = x + jnp.concatenate([jnp.zeros((1, d), x.dtype), x[:, :-d]], axis=1)
        d *= 2
    return x


def _scan_max(x):
    d = 1
    n = x.shape[1]
    while d < n:
        x = jnp.maximum(
            x, jnp.concatenate([jnp.full((1, d), NEG, x.dtype), x[:, :-d]], axis=1))
        d *= 2
    return x


def _mlp_composite_tc(ht, dirs_t, deltas_t, seg2, segp2,
                      w1, b1r, w2, b2c, wr1, br1c, wr2, br2c, wr3, br3c):
    grid = (N // BS,)

    def body(ht_ref, d_ref, de_ref, s_ref, sp_ref,
             w1_ref, b1_ref, w2_ref, b2_ref, wr1_ref, br1_ref,
             wr2_ref, br2_ref, wr3_ref, br3_ref,
             o0_ref, o1_ref, o2_ref, o3_ref, carry):
        pid = pl.program_id(0)

        @pl.when(pid == 0)
        def _():
            carry[0] = 0.0
            carry[1] = NEG

        hts = ht_ref[...]
        hid = jax.nn.relu(
            lax.dot_general(hts, w1_ref[...], (((0,), (0,)), ((), ())),
                            preferred_element_type=jnp.float32) + b1_ref[...])
        g_t = lax.dot_general(w2_ref[...], hid, (((0,), (1,)), ((), ())),
                              preferred_element_type=jnp.float32) + b2_ref[...]
        sigma = jnp.exp(g_t[0:1, :])
        geo = g_t[1:17, :]

        d = d_ref[...]
        nrm = jnp.sqrt(jnp.sum(d * d, axis=0, keepdims=True)) + 1e-8
        dn = d / nrm
        x = dn[0:1, :]
        y = dn[1:2, :]
        z = dn[2:3, :]
        xx = x * x
        yy = y * y
        zz = z * z
        xy = x * y
        yz = y * z
        xz = x * z
        de = jnp.concatenate([
            jnp.full_like(x, SH0),
            -SH1 * y,
            SH1 * z,
            -SH1 * x,
            SH2 * xy,
            -SH2 * yz,
            SH3 * zz - SH4,
            -SH2 * xz,
            SH5 * (xx - yy),
            SH6 * y * (-3.0 * xx + yy),
            SH7 * xy * z,
            SH8 * y * (1.0 - 5.0 * zz),
            SH9 * z * (5.0 * zz - 3.0),
            SH8 * x * (1.0 - 5.0 * zz),
            SH10 * z * (xx - yy),
            SH6 * x * (-xx + 3.0 * yy),
        ], axis=0)
        ri = jnp.concatenate([de, geo], axis=0)
        h2 = jax.nn.relu(
            lax.dot_general(wr1_ref[...], ri, (((0,), (0,)), ((), ())),
                            preferred_element_type=jnp.float32) + br1_ref[...])
        h2 = jax.nn.relu(
            lax.dot_general(wr2_ref[...], h2, (((0,), (0,)), ((), ())),
                            preferred_element_type=jnp.float32) + br2_ref[...])
        rgb = jax.nn.sigmoid(
            lax.dot_general(wr3_ref[...], h2, (((0,), (0,)), ((), ())),
                            preferred_element_type=jnp.float32) + br3_ref[...])

        dlt = de_ref[...]
        dt = dlt[0:1, :] * 0.01
        ts = dlt[1:2, :]
        s = sigma * dt
        c0 = carry[0]
        c1 = carry[1]
        cs = _scan_sum(s)
        excl = (c0 + cs) - s
        bnd = s_ref[...] != sp_ref[...]
        cand = jnp.where(bnd, excl, NEG)
        off = jnp.maximum(_scan_max(cand), c1)
        trans = jnp.exp(-(excl - off))
        alpha = 1.0 - jnp.exp(-s)
        w = alpha * trans
        wrgb = w * rgb
        o0_ref[...] = wrgb[0]
        o1_ref[...] = wrgb[1]
        o2_ref[...] = wrgb[2]
        o3_ref[...] = (w * ts)[0]
        carry[0] = c0 + jnp.sum(s)
        carry[1] = jnp.maximum(jnp.max(cand), c1)

    full = lambda shape: pl.BlockSpec(shape, lambda i: (0, 0))
    blk = lambda r: pl.BlockSpec((r, BS), lambda i: (0, i))
    oblk = pl.BlockSpec((BS,), lambda i: (i,))
    o1d = jax.ShapeDtypeStruct((N,), jnp.float32)
    return pl.pallas_call(
        body,
        grid=grid,
        in_specs=[
            blk(2 * L), blk(3), blk(2), blk(1), blk(1),
            full((2 * L, 64)), full((1, 64)), full((64, 17)), full((17, 1)),
            full((32, 64)), full((64, 1)), full((64, 64)), full((64, 1)),
            full((64, 3)), full((3, 1)),
        ],
        out_specs=[oblk, oblk, oblk, oblk],
        out_shape=[o1d, o1d, o1d, o1d],
        scratch_shapes=[pltpu.SMEM((2,), jnp.float32)],
        compiler_params=pltpu.CompilerParams(
            dimension_semantics=("arbitrary",)),
    )(ht, dirs_t, deltas_t, seg2, segp2,
      w1, b1r, w2, b2c, wr1, br1c, wr2, br2c, wr3, br3c)


CC = 1024
NACC = 4 * N_RAYS


def _segsum_sc(v0a, v1a, v2a, v3a, seg, z1):
    mesh = plsc.VectorSubcoreMesh(core_axis_name="c", subcore_axis_name="s")

    @functools.partial(
        pl.kernel, mesh=mesh,
        out_type=jax.ShapeDtypeStruct((NW * NACC,), jnp.float32),
        scratch_types=[
            pltpu.VMEM((CC,), jnp.int32),
            pltpu.VMEM((CC,), jnp.float32),
            pltpu.VMEM((CC,), jnp.float32),
            pltpu.VMEM((CC,), jnp.float32),
            pltpu.VMEM((CC,), jnp.float32),
            pltpu.VMEM((NACC,), jnp.float32),
        ],
        compiler_params=pltpu.CompilerParams(needs_layout_passes=False),
    )
    def k(v0_h, v1_h, v2_h, v3_h, seg_h, z_h, out_h,
          segv, v0, v1, v2, v3, acc):
        wid = lax.axis_index("s") * 2 + lax.axis_index("c")
        tbase = wid * NT
        pltpu.sync_copy(z_h, acc)

        def chunk_body(ci, _):
            base = tbase + ci * CC
            pltpu.sync_copy(seg_h.at[pl.ds(base, CC)], segv)
            pltpu.sync_copy(v0_h.at[pl.ds(base, CC)], v0)
            pltpu.sync_copy(v1_h.at[pl.ds(base, CC)], v1)
            pltpu.sync_copy(v2_h.at[pl.ds(base, CC)], v2)
            pltpu.sync_copy(v3_h.at[pl.ds(base, CC)], v3)

            def g_body(g, _):
                o = g * 16
                sv = segv[pl.ds(o, 16)]
                plsc.addupdate_scatter(acc, [sv], v0[pl.ds(o, 16)])
                plsc.addupdate_scatter(acc, [sv + N_RAYS], v1[pl.ds(o, 16)])
                plsc.addupdate_scatter(acc, [sv + 2 * N_RAYS], v2[pl.ds(o, 16)])
                plsc.addupdate_scatter(acc, [sv + 3 * N_RAYS], v3[pl.ds(o, 16)])
                return 0

            lax.fori_loop(0, CC // 16, g_body, 0)
            return 0

        lax.fori_loop(0, NT // CC, chunk_body, 0)
        pltpu.sync_copy(acc, out_h.at[pl.ds(wid * NACC, NACC)])

    return k(v0a, v1a, v2a, v3a, seg, z1)


def _reduce_tc(partials):
    def body(p_ref, o_ref):
        o_ref[...] = jnp.sum(p_ref[...], axis=0)

    return pl.pallas_call(
        body,
        out_shape=jax.ShapeDtypeStruct((NACC,), jnp.float32),
    )(partials)


def kernel(xyzs, dirs, deltas, table, w1, b1, w2, b2, wr1, br1, wr2, br2,
           wr3, br3, segment_ids):
    xt = xyzs.T
    xs, ys, zs = xt[0], xt[1], xt[2]
    tab1 = table.reshape(L, T // 128, 128, 2).transpose(0, 1, 3, 2).reshape(
        L * T * 2)
    resf = jnp.asarray(RES, dtype=jnp.float32)

    ht1 = _hash_encode_sc(xs, ys, zs, tab1, resf)
    ht = ht1.reshape(2 * L, N)

    dirs_t = dirs.T
    deltas_t = deltas.T
    seg2 = segment_ids.reshape(1, N)
    segp2 = jnp.concatenate(
        [jnp.full((1,), -1, jnp.int32), segment_ids[:-1]]).reshape(1, N)
    v0a, v1a, v2a, v3a = _mlp_composite_tc(
        ht, dirs_t, deltas_t, seg2, segp2,
        w1, b1.reshape(1, 64), w2, b2.reshape(17, 1),
        wr1, br1.reshape(64, 1), wr2, br2.reshape(64, 1),
        wr3, br3.reshape(3, 1))

    z1 = jnp.zeros((NACC,), jnp.float32)
    partials = _segsum_sc(v0a, v1a, v2a, v3a, segment_ids, z1)
    out4 = _reduce_tc(partials.reshape(NW, NACC)).reshape(4, N_RAYS)
    image = out4[0:3].T
    depth = out4[3]
    return image, depth


# speedup vs baseline: 4.6145x; 1.0291x over previous
import functools
import numpy as np
import jax
import jax.numpy as jnp
from jax import lax
from jax.experimental import pallas as pl
from jax.experimental.pallas import tpu as pltpu
from jax.experimental.pallas import tpu_sc as plsc

L = 16
T = 2 ** 19
N_RAYS = 4096
N = 262144
P1 = np.int32(np.uint32(2654435761).view(np.int32))
P2 = np.int32(805459861)
IMASK = np.int32(T - 1)
HMASK = np.int32(-128)
RES = [float(np.floor(16.0 * 1.3819 ** l)) for l in range(L)]

NW = 32
NT = N // NW
C = 512
NCH = NT // C
NG = C // 16
NEG = -3.4e38


def _hash_encode_sc(xs, ys, zs, tab1, resf):
    mesh = plsc.VectorSubcoreMesh(core_axis_name="c", subcore_axis_name="s")

    @functools.partial(
        pl.kernel, mesh=mesh,
        out_type=jax.ShapeDtypeStruct((2 * L * N,), jnp.float32),
        scratch_types=[
            pltpu.VMEM((C,), jnp.float32),
            pltpu.VMEM((C,), jnp.float32),
            pltpu.VMEM((C,), jnp.float32),
            pltpu.VMEM((C,), jnp.float32),
            pltpu.VMEM((C,), jnp.float32),
            pltpu.VMEM((C,), jnp.float32),
            pltpu.VMEM((C,), jnp.float32),
            pltpu.VMEM((C,), jnp.float32),
            pltpu.VMEM((C,), jnp.float32),
            pltpu.VMEM((16 * C,), jnp.int32),
            pltpu.VMEM((16 * C,), jnp.int32),
            pltpu.VMEM((16 * C,), jnp.float32),
            pltpu.VMEM((16 * C,), jnp.float32),
            pltpu.VMEM((2 * L, C), jnp.float32),
            pltpu.VMEM((L,), jnp.float32),
            pltpu.SemaphoreType.DMA,
            pltpu.SemaphoreType.DMA,
        ],
        compiler_params=pltpu.CompilerParams(needs_layout_passes=False),
    )
    def k(xs_h, ys_h, zs_h, tab_h, res_h, out_h,
          xv, yv, zv, fxa, fya, fza, fxb, fyb, fzb,
          idxa, idxb2, rowa, rowb, hbuf, resv, sem, osem):
        wid = lax.axis_index("s") * 2 + lax.axis_index("c")
        tbase = wid * NT
        pltpu.sync_copy(res_h, resv)

        def chunk_body(ci, _):
            base = tbase + ci * C
            pltpu.sync_copy(xs_h.at[pl.ds(base, C)], xv)
            pltpu.sync_copy(ys_h.at[pl.ds(base, C)], yv)
            pltpu.sync_copy(zs_h.at[pl.ds(base, C)], zv)

            bufA = (fxa, fya, fza, idxa, rowa)
            bufB = (fxb, fyb, fzb, idxb2, rowb)

            def compute_idx(lv, buf):
                fx, fy, fz, idxb, rows = buf
                res = plsc.load_gather(resv, [jnp.full((16,), lv, jnp.int32)])
                lbase2 = lv * (2 * T)

                def idx_body(g, _):
                    o = g * 16
                    xg = xv[pl.ds(o, 16)]
                    yg = yv[pl.ds(o, 16)]
                    zg = zv[pl.ds(o, 16)]
                    px = xg * res
                    py = yg * res
                    pz = zg * res
                    ix = px.astype(jnp.int32)
                    iy = py.astype(jnp.int32)
                    iz = pz.astype(jnp.int32)
                    fx[pl.ds(o, 16)] = px - ix.astype(jnp.float32)
                    fy[pl.ds(o, 16)] = py - iy.astype(jnp.float32)
                    fz[pl.ds(o, 16)] = pz - iz.astype(jnp.float32)
                    hx0 = ix
                    hx1 = ix + 1
                    hy0 = iy * P1
                    hy1 = hy0 + P1
                    hz0 = iz * P2
                    hz1 = hz0 + P2
                    for c in range(8):
                        hx = hx1 if (c & 1) else hx0
                        hy = hy1 if (c & 2) else hy0
                        hz = hz1 if (c & 4) else hz0
                        t = (hx ^ hy ^ hz) & IMASK
                        e = t + (t & HMASK) + lbase2
                        idxb[pl.ds(2 * c * C + o, 16)] = e
                        idxb[pl.ds((2 * c + 1) * C + o, 16)] = e + 128
                    return 0

                lax.fori_loop(0, NG, idx_body, 0)

            def issue(buf):
                return pltpu.async_copy(tab_h.at[buf[3]], buf[4], sem)

            def wait(buf):
                pltpu.make_async_copy(tab_h.at[buf[3]], buf[4], sem).wait()

            def accum(lv, buf):
                fx, fy, fz, idxb, rows = buf

                def acc_body(g, _):
                    o = g * 16
                    fxg = fx[pl.ds(o, 16)]
                    fyg = fy[pl.ds(o, 16)]
                    fzg = fz[pl.ds(o, 16)]
                    gx = 1.0 - fxg
                    gy = 1.0 - fyg
                    gz = 1.0 - fzg
                    h0 = jnp.zeros((16,), jnp.float32)
                    h1 = jnp.zeros((16,), jnp.float32)
                    for c in range(8):
                        wx = fxg if (c & 1) else gx
                        wy = fyg if (c & 2) else gy
                        wz = fzg if (c & 4) else gz
                        wgt = wx * wy * wz
                        f0 = rows[pl.ds(2 * c * C + o, 16)]
                        f1 = rows[pl.ds((2 * c + 1) * C + o, 16)]
                        h0 = h0 + wgt * f0
                        h1 = h1 + wgt * f1
                    hbuf[2 * lv, pl.ds(o, 16)] = h0
                    hbuf[2 * lv + 1, pl.ds(o, 16)] = h1
                    return 0

                lax.fori_loop(0, NG, acc_body, 0)

            compute_idx(0, bufA)
            issue(bufA)

            def level_body(i, _):
                lv = 2 * i
                compute_idx(lv + 1, bufB)
                issue(bufB)
                wait(bufA)
                accum(lv, bufA)

                @pl.when(i < L // 2 - 1)
                def _():
                    compute_idx(lv + 2, bufA)
                    issue(bufA)

                wait(bufB)
                accum(lv + 1, bufB)
                return 0

            lax.fori_loop(0, L // 2, level_body, 0)
            ocps = [
                pltpu.async_copy(hbuf.at[ff],
                                 out_h.at[pl.ds(ff * N + base, C)], osem)
                for ff in range(2 * L)
            ]
            for cp in ocps:
                cp.wait()
            return 0

        lax.fori_loop(0, NCH, chunk_body, 0)

    return k(xs, ys, zs, tab1, resf)


BS = 2048
SH0 = 0.28209479177387814
SH1 = 0.48860251190291987
SH2 = 1.0925484305920792
SH3 = 0.94617469575755997
SH4 = 0.31539156525252005
SH5 = 0.54627421529603959
SH6 = 0.59004358992664352
SH7 = 2.8906114426405538
SH8 = 0.45704579946446572
SH9 = 0.3731763325901154
SH10 = 1.4453057213202769


def _scan_sum(x):
    d = 1
    n = x.shape[1]
    while d < n:
        x = x + jnp.concatenate([jnp.zeros((1, d), x.dtype), x[:, :-d]], axis=1)
        d *= 2
    return x


def _scan_max(x):
    d = 1
    n = x.shape[1]
    while d < n:
        x = jnp.maximum(
            x, jnp.concatenate([jnp.full((1, d), NEG, x.dtype), x[:, :-d]], axis=1))
        d *= 2
    return x


def _mlp_composite_tc(ht, dirs_t, deltas_t, seg2, segp2,
                      w1, b1r, w2, b2c, wr1, br1c, wr2, br2c, wr3, br3c):
    grid = (N // BS,)

    def body(ht_ref, d_ref, de_ref, s_ref, sp_ref,
             w1_ref, b1_ref, w2_ref, b2_ref, wr1_ref, br1_ref,
             wr2_ref, br2_ref, wr3_ref, br3_ref,
             o0_ref, o1_ref, o2_ref, o3_ref, carry):
        pid = pl.program_id(0)

        @pl.when(pid == 0)
        def _():
            carry[0] = 0.0
            carry[1] = NEG

        hts = ht_ref[...]
        hid = jax.nn.relu(
            lax.dot_general(hts, w1_ref[...], (((0,), (0,)), ((), ())),
                            preferred_element_type=jnp.float32) + b1_ref[...])
        g_t = lax.dot_general(w2_ref[...], hid, (((0,), (1,)), ((), ())),
                              preferred_element_type=jnp.float32) + b2_ref[...]
        sigma = jnp.exp(g_t[0:1, :])
        geo = g_t[1:17, :]

        d = d_ref[...]
        nrm = jnp.sqrt(jnp.sum(d * d, axis=0, keepdims=True)) + 1e-8
        dn = d / nrm
        x = dn[0:1, :]
        y = dn[1:2, :]
        z = dn[2:3, :]
        xx = x * x
        yy = y * y
        zz = z * z
        xy = x * y
        yz = y * z
        xz = x * z
        de = jnp.concatenate([
            jnp.full_like(x, SH0),
            -SH1 * y,
            SH1 * z,
            -SH1 * x,
            SH2 * xy,
            -SH2 * yz,
            SH3 * zz - SH4,
            -SH2 * xz,
            SH5 * (xx - yy),
            SH6 * y * (-3.0 * xx + yy),
            SH7 * xy * z,
            SH8 * y * (1.0 - 5.0 * zz),
            SH9 * z * (5.0 * zz - 3.0),
            SH8 * x * (1.0 - 5.0 * zz),
            SH10 * z * (xx - yy),
            SH6 * x * (-xx + 3.0 * yy),
        ], axis=0)
        ri = jnp.concatenate([de, geo], axis=0)
        h2 = jax.nn.relu(
            lax.dot_general(wr1_ref[...], ri, (((0,), (0,)), ((), ())),
                            preferred_element_type=jnp.float32) + br1_ref[...])
        h2 = jax.nn.relu(
            lax.dot_general(wr2_ref[...], h2, (((0,), (0,)), ((), ())),
                            preferred_element_type=jnp.float32) + br2_ref[...])
        rgb = jax.nn.sigmoid(
            lax.dot_general(wr3_ref[...], h2, (((0,), (0,)), ((), ())),
                            preferred_element_type=jnp.float32) + br3_ref[...])

        dlt = de_ref[...]
        dt = dlt[0:1, :] * 0.01
        ts = dlt[1:2, :]
        s = sigma * dt
        c0 = carry[0]
        c1 = carry[1]
        cs = _scan_sum(s)
        excl = (c0 + cs) - s
        bnd = s_ref[...] != sp_ref[...]
        cand = jnp.where(bnd, excl, NEG)
        off = jnp.maximum(_scan_max(cand), c1)
        trans = jnp.exp(-(excl - off))
        alpha = 1.0 - jnp.exp(-s)
        w = alpha * trans
        wrgb = w * rgb
        o0_ref[...] = wrgb[0]
        o1_ref[...] = wrgb[1]
        o2_ref[...] = wrgb[2]
        o3_ref[...] = (w * ts)[0]
        carry[0] = c0 + jnp.sum(s)
        carry[1] = jnp.maximum(jnp.max(cand), c1)

    full = lambda shape: pl.BlockSpec(shape, lambda i: (0, 0))
    blk = lambda r: pl.BlockSpec((r, BS), lambda i: (0, i))
    oblk = pl.BlockSpec((BS,), lambda i: (i,))
    o1d = jax.ShapeDtypeStruct((N,), jnp.float32)
    return pl.pallas_call(
        body,
        grid=grid,
        in_specs=[
            blk(2 * L), blk(3), blk(2), blk(1), blk(1),
            full((2 * L, 64)), full((1, 64)), full((64, 17)), full((17, 1)),
            full((32, 64)), full((64, 1)), full((64, 64)), full((64, 1)),
            full((64, 3)), full((3, 1)),
        ],
        out_specs=[oblk, oblk, oblk, oblk],
        out_shape=[o1d, o1d, o1d, o1d],
        scratch_shapes=[pltpu.SMEM((2,), jnp.float32)],
        compiler_params=pltpu.CompilerParams(
            dimension_semantics=("arbitrary",)),
    )(ht, dirs_t, deltas_t, seg2, segp2,
      w1, b1r, w2, b2c, wr1, br1c, wr2, br2c, wr3, br3c)


CC = 1024
NACC = 4 * N_RAYS


def _segsum_sc(v0a, v1a, v2a, v3a, seg, z1):
    mesh = plsc.VectorSubcoreMesh(core_axis_name="c", subcore_axis_name="s")

    @functools.partial(
        pl.kernel, mesh=mesh,
        out_type=jax.ShapeDtypeStruct((NW * NACC,), jnp.float32),
        scratch_types=[
            pltpu.VMEM((CC,), jnp.int32),
            pltpu.VMEM((CC,), jnp.float32),
            pltpu.VMEM((CC,), jnp.float32),
            pltpu.VMEM((CC,), jnp.float32),
            pltpu.VMEM((CC,), jnp.float32),
            pltpu.VMEM((NACC,), jnp.float32),
        ],
        compiler_params=pltpu.CompilerParams(needs_layout_passes=False),
    )
    def k(v0_h, v1_h, v2_h, v3_h, seg_h, z_h, out_h,
          segv, v0, v1, v2, v3, acc):
        wid = lax.axis_index("s") * 2 + lax.axis_index("c")
        tbase = wid * NT
        pltpu.sync_copy(z_h, acc)

        def chunk_body(ci, _):
            base = tbase + ci * CC
            pltpu.sync_copy(seg_h.at[pl.ds(base, CC)], segv)
            pltpu.sync_copy(v0_h.at[pl.ds(base, CC)], v0)
            pltpu.sync_copy(v1_h.at[pl.ds(base, CC)], v1)
            pltpu.sync_copy(v2_h.at[pl.ds(base, CC)], v2)
            pltpu.sync_copy(v3_h.at[pl.ds(base, CC)], v3)

            def g_body(g, _):
                o = g * 16
                sv = segv[pl.ds(o, 16)]
                plsc.addupdate_scatter(acc, [sv], v0[pl.ds(o, 16)])
                plsc.addupdate_scatter(acc, [sv + N_RAYS], v1[pl.ds(o, 16)])
                plsc.addupdate_scatter(acc, [sv + 2 * N_RAYS], v2[pl.ds(o, 16)])
                plsc.addupdate_scatter(acc, [sv + 3 * N_RAYS], v3[pl.ds(o, 16)])
                return 0

            lax.fori_loop(0, CC // 16, g_body, 0)
            return 0

        lax.fori_loop(0, NT // CC, chunk_body, 0)
        pltpu.sync_copy(acc, out_h.at[pl.ds(wid * NACC, NACC)])

    return k(v0a, v1a, v2a, v3a, seg, z1)


def _reduce_tc(partials):
    def body(p_ref, o_ref):
        o_ref[...] = jnp.sum(p_ref[...], axis=0)

    return pl.pallas_call(
        body,
        out_shape=jax.ShapeDtypeStruct((NACC,), jnp.float32),
    )(partials)


def kernel(xyzs, dirs, deltas, table, w1, b1, w2, b2, wr1, br1, wr2, br2,
           wr3, br3, segment_ids):
    xt = xyzs.T
    xs, ys, zs = xt[0], xt[1], xt[2]
    tab1 = table.reshape(L, T // 128, 128, 2).transpose(0, 1, 3, 2).reshape(
        L * T * 2)
    resf = jnp.asarray(RES, dtype=jnp.float32)

    ht1 = _hash_encode_sc(xs, ys, zs, tab1, resf)
    ht = ht1.reshape(2 * L, N)

    dirs_t = dirs.T
    deltas_t = deltas.T
    seg2 = segment_ids.reshape(1, N)
    segp2 = jnp.concatenate(
        [jnp.full((1,), -1, jnp.int32), segment_ids[:-1]]).reshape(1, N)
    v0a, v1a, v2a, v3a = _mlp_composite_tc(
        ht, dirs_t, deltas_t, seg2, segp2,
        w1, b1.reshape(1, 64), w2, b2.reshape(17, 1),
        wr1, br1.reshape(64, 1), wr2, br2.reshape(64, 1),
        wr3, br3.reshape(3, 1))

    z1 = jnp.zeros((NACC,), jnp.float32)
    partials = _segsum_sc(v0a, v1a, v2a, v3a, segment_ids, z1)
    out4 = _reduce_tc(partials.reshape(NW, NACC)).reshape(4, N_RAYS)
    image = out4[0:3].T
    depth = out4[3]
    return image, depth
